# baseline (speedup 1.0000x reference)
import jax
import jax.numpy as jnp
from jax.experimental import pallas as pl
from jax.experimental.pallas import tpu as pltpu


def _repel_kernel(ut_ref, out_ref):
    ut = ut_ref[...]
    inv_sq = pl.reciprocal(ut * ut, approx=True)
    s = jnp.sum(inv_sq, axis=0, keepdims=True)
    rows = out_ref.shape[0]
    stacked = jnp.concatenate(
        [s[:, k * 128:(k + 1) * 128] for k in range(rows)], axis=0)
    out_ref[...] = stacked * jnp.float32(71.0)


def kernel(U):
    n, d = U.shape
    orig_dtype = U.dtype

    block_n = 131072
    num_blocks = pl.cdiv(n, block_n)
    padded_n = num_blocks * block_n

    ut = U.astype(jnp.float32).T
    if padded_n != n:
        ut = jnp.concatenate(
            [ut, jnp.ones((d, padded_n - n), jnp.float32)], axis=1)

    rows_per_block = block_n // 128
    out = pl.pallas_call(
        _repel_kernel,
        out_shape=jax.ShapeDtypeStruct((padded_n // 128, 128), jnp.float32),
        grid=(num_blocks,),
        in_specs=[pl.BlockSpec((d, block_n), lambda i: (0, i))],
        out_specs=pl.BlockSpec((rows_per_block, 128), lambda i: (i, 0)),
        compiler_params=pltpu.CompilerParams(
            dimension_semantics=("parallel",),
            vmem_limit_bytes=100 * 1024 * 1024,
        ),
    )(ut)

    return out.reshape(padded_n, 1)[:n].astype(orig_dtype)


# speedup vs baseline: 1.0804x; 1.0804x over previous
import jax
import jax.numpy as jnp
from jax.experimental import pallas as pl
from jax.experimental.pallas import tpu as pltpu


def _repel_kernel(ut_ref, out_ref):
    ut = ut_ref[...]
    inv_sq = pl.reciprocal(ut * ut, approx=True)
    s = jnp.sum(inv_sq, axis=0, keepdims=True)
    rows = out_ref.shape[0]
    stacked = jnp.concatenate(
        [s[:, k * 128:(k + 1) * 128] for k in range(rows)], axis=0)
    out_ref[...] = stacked * jnp.float32(71.0)


def kernel(U):
    n, d = U.shape
    orig_dtype = U.dtype

    block_n = 65536
    num_blocks = pl.cdiv(n, block_n)
    padded_n = num_blocks * block_n

    ut = U.astype(jnp.float32).T
    if padded_n != n:
        ut = jnp.concatenate(
            [ut, jnp.ones((d, padded_n - n), jnp.float32)], axis=1)

    rows_per_block = block_n // 128
    out = pl.pallas_call(
        _repel_kernel,
        out_shape=jax.ShapeDtypeStruct((padded_n // 128, 128), jnp.float32),
        grid=(num_blocks,),
        in_specs=[pl.BlockSpec((d, block_n), lambda i: (0, i))],
        out_specs=pl.BlockSpec((rows_per_block, 128), lambda i: (i, 0)),
        compiler_params=pltpu.CompilerParams(
            dimension_semantics=("parallel",),
            vmem_limit_bytes=100 * 1024 * 1024,
        ),
    )(ut)

    return out.reshape(padded_n, 1)[:n].astype(orig_dtype)
